# baseline (speedup 1.0000x reference)
import functools

import jax
import jax.numpy as jnp
from jax import lax
from jax.experimental import pallas as pl
from jax.experimental.pallas import tpu as pltpu
from jax.experimental.pallas import tpu_sc as plsc

N = 10000
D = 128
E = 320000

NC = 2
NS = 16
NW = NC * NS
EPT = E // NW
C = 40
NCHUNK = EPT // C
NB = 5
RA = 624
NG = D // 16


def _spmm_body(table, packed, vals2, out, b0, b1, b2, b3, b4,
               i0, i1, i2, i3, i4, v0, v1, v2, v3, v4, acc_sh,
               gs0, gs1, gs2, gs3, gs4, ss0, ss1, ss2, ss3, ss4,
               is0, is1, is2, is3, is4, zsem):
  c = lax.axis_index("c")
  s = lax.axis_index("s")
  wid = c * NS + s
  bufs = (b0, b1, b2, b3, b4)
  ibs = (i0, i1, i2, i3, i4)
  vbs = (v0, v1, v2, v3, v4)
  gsem = (gs0, gs1, gs2, gs3, gs4)
  ssem = (ss0, ss1, ss2, ss3, ss4)
  isem = (is0, is1, is2, is3, is4)

  pltpu.async_copy(packed.at[wid, 0], ibs[0], isem[0])
  pltpu.async_copy(vals2.at[wid, 0], vbs[0], isem[0])
  pltpu.async_copy(packed.at[wid, 1], ibs[1], isem[1])
  pltpu.async_copy(vals2.at[wid, 1], vbs[1], isem[1])

  zv = jnp.zeros((16,), jnp.float32)
  def _zero(i, _):
    for j in range(NG):
      b4[i, pl.ds(j * 16, 16)] = zv
    return 0
  lax.fori_loop(0, C, _zero, 0)

  pltpu.make_async_copy(packed.at[wid, 0], ibs[0], isem[0]).wait()
  pltpu.make_async_copy(vals2.at[wid, 0], vbs[0], isem[0]).wait()
  pltpu.async_copy(table.at[ibs[0].at[0]], bufs[0], gsem[0])

  def _zcopy(k, _):
    pltpu.async_copy(b4, acc_sh.at[pl.ds(s * RA + k * C, C)], zsem)
    return 0
  lax.fori_loop(0, 15, _zcopy, 0)
  @pl.when(s < NS - 1)
  def _ztail0():
    pltpu.async_copy(b4.at[pl.ds(0, RA - 15 * C)],
                     acc_sh.at[pl.ds(s * RA + 15 * C, RA - 15 * C)], zsem)
  @pl.when(s == NS - 1)
  def _ztail1():
    pltpu.async_copy(b4, acc_sh.at[pl.ds(s * RA + 15 * C, C)], zsem)
  def _zdrain(k, _):
    pltpu.make_async_copy(b4, acc_sh.at[pl.ds(0, C)], zsem).wait()
    return 0
  lax.fori_loop(0, 15, _zdrain, 0)
  @pl.when(s < NS - 1)
  def _zdrain0():
    pltpu.make_async_copy(b4.at[pl.ds(0, RA - 15 * C)],
                          acc_sh.at[pl.ds(0, RA - 15 * C)], zsem).wait()
  @pl.when(s == NS - 1)
  def _zdrain1():
    pltpu.make_async_copy(b4, acc_sh.at[pl.ds(0, C)], zsem).wait()
  plsc.subcore_barrier()

  def _group(g, _):
    for i in range(NB):
      k = g * NB + i
      m = (i + 2) % NB
      n = (i + 1) % NB
      pltpu.make_async_copy(table.at[ibs[0].at[0]], bufs[i], gsem[i]).wait()
      @pl.when((k + 2 < NCHUNK) & (k >= NB - 2))
      def _ws():
        pltpu.make_async_copy(bufs[m], acc_sh.at[ibs[0].at[1]],
                              ssem[m]).wait()
      @pl.when(k + 2 < NCHUNK)
      def _in():
        pltpu.async_copy(packed.at[wid, k + 2], ibs[m], isem[m])
        pltpu.async_copy(vals2.at[wid, k + 2], vbs[m], isem[m])
      @pl.when(k + 1 < NCHUNK)
      def _gn():
        pltpu.make_async_copy(packed.at[wid, 0], ibs[n], isem[n]).wait()
        pltpu.make_async_copy(vals2.at[wid, 0], vbs[n], isem[n]).wait()
        pltpu.async_copy(table.at[ibs[n].at[0]], bufs[n], gsem[n])
      for base, lo in ((0, 0), (16, 0), (24, 8)):
        vg = vbs[i][pl.ds(base, 16)]
        for t in range(lo, 16):
          e = base + t
          v = vg[t]
          for j in range(NG):
            bufs[i][e, pl.ds(j * 16, 16)] = bufs[i][e, pl.ds(j * 16, 16)] * v
      pltpu.async_copy(bufs[i], acc_sh.at[ibs[i].at[1]], ssem[i], add=True)
    return 0
  lax.fori_loop(0, NCHUNK // NB, _group, 0)
  for i in range(NB):
    pltpu.make_async_copy(bufs[i], acc_sh.at[ibs[0].at[1]], ssem[i]).wait()
  plsc.subcore_barrier()

  pltpu.sync_copy(acc_sh.at[pl.ds(s * RA, RA)], out.at[c, pl.ds(s * RA, RA)])
  @pl.when(s == NS - 1)
  def _wtail():
    pltpu.sync_copy(acc_sh.at[pl.ds(NS * RA, N - NS * RA)],
                    out.at[c, pl.ds(NS * RA, N - NS * RA)])


_sc_spmm = pl.kernel(
    _spmm_body,
    out_type=jax.ShapeDtypeStruct((NC, N, D), jnp.float32),
    mesh=plsc.VectorSubcoreMesh(core_axis_name="c", subcore_axis_name="s"),
    scratch_types=(
        [pltpu.VMEM((C, D), jnp.float32)] * NB +
        [pltpu.VMEM((2, C), jnp.int32)] * NB +
        [pltpu.VMEM((C,), jnp.float32)] * NB + [
            pltpu.VMEM_SHARED((N, D), jnp.float32),
        ] + [pltpu.SemaphoreType.DMA] * (3 * NB + 1)
    ),
)


BR = 1000


def _combine_body(p_ref, o_ref):
  o_ref[...] = p_ref[0] + p_ref[1]


def _tc_combine(p):
  return pl.pallas_call(
      _combine_body,
      grid=(N // BR,),
      in_specs=[pl.BlockSpec((NC, BR, D), lambda i: (0, i, 0))],
      out_specs=pl.BlockSpec((BR, D), lambda i: (i, 0)),
      out_shape=jax.ShapeDtypeStruct((N, D), jnp.float32),
  )(p)


def _finish_body(x_ref, p_ref, w_ref, b_ref, o_ref):
  xb = x_ref[...]
  s = jnp.sum(xb * w_ref[0][None, :], axis=1, keepdims=True) + b_ref[0, 0]
  a = jnp.clip(jax.nn.sigmoid(s), 1e-6, 1.0 - 1e-6)
  f = a * (p_ref[0] + p_ref[1]) - xb
  f = jnp.where(jnp.isnan(f), 0.0, jnp.clip(f, -1e6, 1e6))
  o_ref[...] = f


def _tc_finish(x, p, W, b):
  b2 = b.reshape(1, 1)
  return pl.pallas_call(
      _finish_body,
      grid=(N // BR,),
      in_specs=[
          pl.BlockSpec((BR, D), lambda i: (i, 0)),
          pl.BlockSpec((NC, BR, D), lambda i: (0, i, 0)),
          pl.BlockSpec((1, D), lambda i: (0, 0)),
          pl.BlockSpec((1, 1), lambda i: (0, 0)),
      ],
      out_specs=pl.BlockSpec((BR, D), lambda i: (i, 0)),
      out_shape=jax.ShapeDtypeStruct((N, D), jnp.float32),
  )(x, p, W, b2)


@jax.jit
def _run(x, packed, vals2, W, b):
  p1 = _sc_spmm(x, packed, vals2)
  ax1 = _tc_combine(p1)
  p2 = _sc_spmm(ax1, packed, vals2)
  return _tc_finish(x, p2, W, b)


def kernel(t, x, edge_index, adj_vals, W, b):
  del t
  col3 = edge_index[1].reshape(NW, NCHUNK, 1, C)
  row3 = edge_index[0].reshape(NW, NCHUNK, 1, C)
  packed = jnp.concatenate([col3, row3], axis=2)
  vals2 = adj_vals.reshape(NW, NCHUNK, C)
  return _run(x, packed, vals2, W, b)


# speedup vs baseline: 1.7916x; 1.7916x over previous
import functools

import jax
import jax.numpy as jnp
from jax import lax
from jax.experimental import pallas as pl
from jax.experimental.pallas import tpu as pltpu
from jax.experimental.pallas import tpu_sc as plsc

N = 10000
D = 128
E = 320000

NC = 2
NS = 16
NW = NC * NS
EPT = E // NW
C = 80
NCHUNK = EPT // C
NB = 4
NMAIN = NCHUNK - 1
RA = 624
NG = D // 16


def _spmm_body(table, packed, vals2, out, b0, b1, b2, b3,
               i0, i1, i2, i3, v0, v1, v2, v3, acc_sh,
               gs0, gs1, gs2, gs3, ss0, ss1, ss2, ss3,
               is0, is1, is2, is3, zsem):
  c = lax.axis_index("c")
  s = lax.axis_index("s")
  wid = c * NS + s
  bufs = (b0, b1, b2, b3)
  ibs = (i0, i1, i2, i3)
  vbs = (v0, v1, v2, v3)
  gsem = (gs0, gs1, gs2, gs3)
  ssem = (ss0, ss1, ss2, ss3)
  isem = (is0, is1, is2, is3)

  def idx_load(j, slot):
    pltpu.async_copy(packed.at[wid, j], ibs[slot], isem[slot])
    pltpu.async_copy(vals2.at[wid, j], vbs[slot], isem[slot])

  def idx_wait(slot):
    pltpu.make_async_copy(packed.at[wid, 0], ibs[slot], isem[slot]).wait()
    pltpu.make_async_copy(vals2.at[wid, 0], vbs[slot], isem[slot]).wait()

  def gather(slot):
    pltpu.async_copy(table.at[ibs[slot].at[0]], bufs[slot], gsem[slot])

  def gather_wait(slot):
    pltpu.make_async_copy(table.at[ibs[0].at[0]], bufs[slot],
                          gsem[slot]).wait()

  def scatter(slot):
    pltpu.async_copy(bufs[slot], acc_sh.at[ibs[slot].at[1]], ssem[slot],
                     add=True)

  def scatter_wait(slot):
    pltpu.make_async_copy(bufs[slot], acc_sh.at[ibs[0].at[1]],
                          ssem[slot]).wait()

  def scale(slot):
    def _scale(gg, _):
      vg = vbs[slot][pl.ds(gg * 16, 16)]
      for t in range(16):
        e = gg * 16 + t
        v = vg[t]
        for j in range(NG):
          bufs[slot][e, pl.ds(j * 16, 16)] = (
              bufs[slot][e, pl.ds(j * 16, 16)] * v)
      return 0
    lax.fori_loop(0, C // 16, _scale, 0)

  idx_load(0, 0)
  idx_load(1, 1)

  zv = jnp.zeros((16,), jnp.float32)
  def _zero(i, _):
    for j in range(NG):
      b3[i, pl.ds(j * 16, 16)] = zv
    return 0
  lax.fori_loop(0, C, _zero, 0)

  idx_wait(0)
  gather(0)
  idx_wait(1)
  gather(1)

  ZCH = RA // C
  def _zcopy(k, _):
    pltpu.async_copy(b3, acc_sh.at[pl.ds(s * RA + k * C, C)], zsem)
    return 0
  lax.fori_loop(0, ZCH, _zcopy, 0)
  @pl.when(s < NS - 1)
  def _ztail0():
    pltpu.async_copy(b3.at[pl.ds(0, RA - ZCH * C)],
                     acc_sh.at[pl.ds(s * RA + ZCH * C, RA - ZCH * C)], zsem)
  @pl.when(s == NS - 1)
  def _ztail1():
    pltpu.async_copy(b3, acc_sh.at[pl.ds(s * RA + ZCH * C, C)], zsem)
  def _zdrain(k, _):
    pltpu.make_async_copy(b3, acc_sh.at[pl.ds(0, C)], zsem).wait()
    return 0
  lax.fori_loop(0, ZCH, _zdrain, 0)
  @pl.when(s < NS - 1)
  def _zdrain0():
    pltpu.make_async_copy(b3.at[pl.ds(0, RA - ZCH * C)],
                          acc_sh.at[pl.ds(0, RA - ZCH * C)], zsem).wait()
  @pl.when(s == NS - 1)
  def _zdrain1():
    pltpu.make_async_copy(b3, acc_sh.at[pl.ds(0, C)], zsem).wait()
  plsc.subcore_barrier()

  def _group(g, _):
    for i in range(NB):
      k = g * NB + i
      m2 = (i + 2) % NB
      @pl.when((k + 2 < NCHUNK) & (k >= 2))
      def _ws():
        scatter_wait(m2)
      @pl.when(k + 2 < NCHUNK)
      def _in():
        idx_load(k + 2, m2)
      gather_wait(i)
      scale(i)
      @pl.when(k + 2 < NCHUNK)
      def _gn():
        idx_wait(m2)
        gather(m2)
      scatter(i)
    return 0
  lax.fori_loop(0, NMAIN // NB, _group, 0)
  ti = NMAIN % NB
  gather_wait(ti)
  scale(ti)
  scatter(ti)
  for i in range(NB):
    scatter_wait(i)
  plsc.subcore_barrier()

  pltpu.sync_copy(acc_sh.at[pl.ds(s * RA, RA)], out.at[c, pl.ds(s * RA, RA)])
  @pl.when(s == NS - 1)
  def _wtail():
    pltpu.sync_copy(acc_sh.at[pl.ds(NS * RA, N - NS * RA)],
                    out.at[c, pl.ds(NS * RA, N - NS * RA)])


_sc_spmm = pl.kernel(
    _spmm_body,
    out_type=jax.ShapeDtypeStruct((NC, N, D), jnp.float32),
    mesh=plsc.VectorSubcoreMesh(core_axis_name="c", subcore_axis_name="s"),
    scratch_types=(
        [pltpu.VMEM((C, D), jnp.float32)] * NB +
        [pltpu.VMEM((2, C), jnp.int32)] * NB +
        [pltpu.VMEM((C,), jnp.float32)] * NB + [
            pltpu.VMEM_SHARED((N, D), jnp.float32),
        ] + [pltpu.SemaphoreType.DMA] * (3 * NB + 1)
    ),
)


BR = 1000


def _combine_body(p_ref, o_ref):
  o_ref[...] = p_ref[0] + p_ref[1]


def _tc_combine(p):
  return pl.pallas_call(
      _combine_body,
      grid=(N // BR,),
      in_specs=[pl.BlockSpec((NC, BR, D), lambda i: (0, i, 0))],
      out_specs=pl.BlockSpec((BR, D), lambda i: (i, 0)),
      out_shape=jax.ShapeDtypeStruct((N, D), jnp.float32),
  )(p)


def _finish_body(x_ref, p_ref, w_ref, b_ref, o_ref):
  xb = x_ref[...]
  s = jnp.sum(xb * w_ref[0][None, :], axis=1, keepdims=True) + b_ref[0, 0]
  a = jnp.clip(jax.nn.sigmoid(s), 1e-6, 1.0 - 1e-6)
  f = a * (p_ref[0] + p_ref[1]) - xb
  f = jnp.where(jnp.isnan(f), 0.0, jnp.clip(f, -1e6, 1e6))
  o_ref[...] = f


def _tc_finish(x, p, W, b):
  b2 = b.reshape(1, 1)
  return pl.pallas_call(
      _finish_body,
      grid=(N // BR,),
      in_specs=[
          pl.BlockSpec((BR, D), lambda i: (i, 0)),
          pl.BlockSpec((NC, BR, D), lambda i: (0, i, 0)),
          pl.BlockSpec((1, D), lambda i: (0, 0)),
          pl.BlockSpec((1, 1), lambda i: (0, 0)),
      ],
      out_specs=pl.BlockSpec((BR, D), lambda i: (i, 0)),
      out_shape=jax.ShapeDtypeStruct((N, D), jnp.float32),
  )(x, p, W, b2)


@jax.jit
def _run(x, packed, vals2, W, b):
  p1 = _sc_spmm(x, packed, vals2)
  ax1 = _tc_combine(p1)
  p2 = _sc_spmm(ax1, packed, vals2)
  return _tc_finish(x, p2, W, b)


def kernel(t, x, edge_index, adj_vals, W, b):
  del t
  col3 = edge_index[1].reshape(NW, NCHUNK, 1, C)
  row3 = edge_index[0].reshape(NW, NCHUNK, 1, C)
  packed = jnp.concatenate([col3, row3], axis=2)
  vals2 = adj_vals.reshape(NW, NCHUNK, C)
  return _run(x, packed, vals2, W, b)
